# Initial kernel scaffold; baseline (speedup 1.0000x reference)
import jax
import jax.numpy as jnp
from jax.experimental import pallas as pl


def kernel(p0, p1, p2, Wc, bc, C):
    raise NotImplementedError("write your pallas kernel here")


# speedup vs baseline: 16.7895x; 16.7895x over previous
import jax
import jax.numpy as jnp
from jax.experimental import pallas as pl
from jax.experimental.pallas import tpu as pltpu

DIM = 1792
SCALE = 56
HW = SCALE * SCALE
N_CENTERS = 3136
B = 2
ROWS = B * HW
R = 224
NB = ROWS // R


def _avg_pool3(x):
    s = jax.lax.reduce_window(x, 0.0, jax.lax.add, (1, 1, 3, 3), (1, 1, 1, 1), 'SAME')
    return s / 9.0


def _fused_body(xin_ref, coords_ref, wct_ref, wxy_ref, bc_ref, c_ref,
                out_ref, c2_ref):
    @pl.when(pl.program_id(0) == 0)
    def _():
        cf = c_ref[...].astype(jnp.float32)
        c2_ref[...] = jnp.sum(cf * cf, axis=0, keepdims=True)

    phi = jnp.dot(xin_ref[...], wct_ref[...],
                  preferred_element_type=jnp.float32)
    coords = coords_ref[...]
    wxy = wxy_ref[...]
    phi = (phi + coords[:, 0:1] * wxy[0:1, :]
           + coords[:, 1:2] * wxy[1:2, :] + bc_ref[...])

    f = jnp.sum(phi * phi, axis=1, keepdims=True)
    d = jnp.dot(phi.astype(jnp.bfloat16), c_ref[...],
                preferred_element_type=jnp.float32)
    dist2 = f + c2_ref[...] - 2.0 * d

    iota = jax.lax.broadcasted_iota(jnp.int32, dist2.shape, 1)
    m0 = jnp.min(dist2, axis=1, keepdims=True)
    i0 = jnp.argmin(dist2, axis=1)[:, None]
    d1m = jnp.where(iota == i0, jnp.inf, dist2)
    m1 = jnp.min(d1m, axis=1, keepdims=True)
    i1 = jnp.argmin(d1m, axis=1)[:, None]
    d2m = jnp.where(iota == i1, jnp.inf, d1m)
    m2 = jnp.min(d2m, axis=1, keepdims=True)

    d0 = jnp.sqrt(m0)
    d1 = jnp.sqrt(m1)
    d2 = jnp.sqrt(m2)
    sm0 = 1.0 / (1.0 + jnp.exp(d0 - d1) + jnp.exp(d0 - d2))
    out_ref[...] = d0 * sm0


def kernel(p0, p1, p2, Wc, bc, C):
    s0 = _avg_pool3(p0)
    s1 = jax.image.resize(_avg_pool3(p1), (B, p1.shape[1], SCALE, SCALE),
                          method='bilinear')
    s2 = jax.image.resize(_avg_pool3(p2), (B, p2.shape[1], SCALE, SCALE),
                          method='bilinear')
    sample = jnp.concatenate([s0, s1, s2], axis=1)
    xin = jnp.transpose(sample, (0, 2, 3, 1)).reshape(ROWS, DIM)
    xin = xin.astype(jnp.bfloat16)

    ax = (jnp.arange(SCALE, dtype=jnp.float32) / (SCALE - 1)) * 2.0 - 1.0
    xx = jnp.tile(ax, SCALE)
    yy = jnp.repeat(ax, SCALE)
    coords_hw = jnp.stack([xx, yy], axis=1)
    coords = jnp.concatenate([coords_hw] * B, axis=0)

    wct = jnp.transpose(Wc[:, :DIM]).astype(jnp.bfloat16)
    wxy = jnp.transpose(Wc[:, DIM:])
    bc2 = bc.reshape(1, DIM)
    cb = C.astype(jnp.bfloat16)

    score = pl.pallas_call(
        _fused_body,
        grid=(NB,),
        in_specs=[
            pl.BlockSpec((R, DIM), lambda i: (i, 0)),
            pl.BlockSpec((R, 2), lambda i: (i, 0)),
            pl.BlockSpec((DIM, DIM), lambda i: (0, 0)),
            pl.BlockSpec((2, DIM), lambda i: (0, 0)),
            pl.BlockSpec((1, DIM), lambda i: (0, 0)),
            pl.BlockSpec((DIM, N_CENTERS), lambda i: (0, 0)),
        ],
        out_specs=pl.BlockSpec((R, 1), lambda i: (i, 0)),
        out_shape=jax.ShapeDtypeStruct((ROWS, 1), jnp.float32),
        scratch_shapes=[pltpu.VMEM((1, N_CENTERS), jnp.float32)],
    )(xin, coords, wct, wxy, bc2, cb)

    return score.reshape(B, 1, SCALE, SCALE)
